# Initial kernel scaffold; baseline (speedup 1.0000x reference)
import jax
import jax.numpy as jnp
from jax.experimental import pallas as pl


def kernel(x, edge_index, weight, bias):
    raise NotImplementedError("write your pallas kernel here")


# speedup vs baseline: 8.3127x; 8.3127x over previous
import functools

import jax
import jax.numpy as jnp
from jax import lax
from jax.experimental import pallas as pl
from jax.experimental.pallas import tpu as pltpu
from jax.experimental.pallas import tpu_sc as plsc

N_NODES = 10000
N_EDGES = 320000
D = 128

NC = 2
NS = 16
NW = NC * NS
EPT = N_EDGES // NW
K = 80
CH = EPT // K
SEG = 25
NSEG = CH // SEG
NPAD = 10240
RPT = NPAD // NS
DW = 16

_mesh = plsc.VectorSubcoreMesh(core_axis_name="c", subcore_axis_name="s")


@functools.partial(
    pl.kernel,
    out_type=[
        jax.ShapeDtypeStruct((NC, NPAD, D), jnp.float32),
        jax.ShapeDtypeStruct((NC, NPAD, DW), jnp.float32),
    ],
    mesh=_mesh,
    scratch_types=[
        pltpu.VMEM((SEG, K), jnp.int32),
        pltpu.VMEM((SEG, K), jnp.int32),
        pltpu.VMEM((K, D), jnp.float32),
        pltpu.VMEM((K, DW), jnp.float32),
        pltpu.VMEM((K, DW), jnp.float32),
        pltpu.VMEM_SHARED((NPAD, D), jnp.float32),
        pltpu.VMEM_SHARED((NPAD, DW), jnp.float32),
        pltpu.SemaphoreType.DMA,
    ],
    compiler_params=pltpu.CompilerParams(use_tc_tiling_on_sc=False),
)
def _sc_aggregate(x_hbm, src_hbm, dst_hbm, out_sum, out_deg,
                  idx_s, idx_d, rows, ones_v, zdeg,
                  acc_sh, deg_sh, sem):
    cid = lax.axis_index("c")
    sid = lax.axis_index("s")
    wid = cid * NS + sid

    z16 = jnp.zeros((16,), jnp.float32)
    o16 = jnp.ones((16,), jnp.float32)

    def _zrow(r, _):
        for c in range(D // 16):
            rows[r, pl.ds(c * 16, 16)] = z16
        ones_v[r, :] = o16
        zdeg[r, :] = z16
        return 0

    lax.fori_loop(0, K, _zrow, 0)

    for b in range(RPT // K):
        pltpu.sync_copy(rows, acc_sh.at[pl.ds(sid * RPT + b * K, K)])
        pltpu.sync_copy(zdeg, deg_sh.at[pl.ds(sid * RPT + b * K, K)])
    plsc.subcore_barrier()

    for s in range(NSEG):
        pltpu.sync_copy(src_hbm.at[wid, pl.ds(s * SEG, SEG)], idx_s)
        pltpu.sync_copy(dst_hbm.at[wid, pl.ds(s * SEG, SEG)], idx_d)

        def _chunk(j, _):
            pltpu.async_copy(x_hbm.at[idx_s.at[j]], rows, sem).wait()
            pltpu.sync_copy(rows, acc_sh.at[idx_d.at[j]], add=True)
            pltpu.sync_copy(ones_v, deg_sh.at[idx_d.at[j]], add=True)
            return 0

        lax.fori_loop(0, SEG, _chunk, 0)
    plsc.subcore_barrier()

    pltpu.sync_copy(acc_sh.at[pl.ds(sid * RPT, RPT)],
                    out_sum.at[cid, pl.ds(sid * RPT, RPT)])
    pltpu.sync_copy(deg_sh.at[pl.ds(sid * RPT, RPT)],
                    out_deg.at[cid, pl.ds(sid * RPT, RPT)])


_R = 1000


def _tc_body(x_ref, ps_ref, pd_ref, w_ref, b_ref, out_ref):
    s = ps_ref[0] + ps_ref[1]
    d = pd_ref[0, :, 0:1] + pd_ref[1, :, 0:1]
    agg = s / jnp.maximum(d, 1.0)
    h = jnp.dot(x_ref[...], w_ref[0:D, :], preferred_element_type=jnp.float32)
    h += jnp.dot(agg, w_ref[D:2 * D, :], preferred_element_type=jnp.float32)
    nrm = jnp.sqrt(jnp.sum(h * h, axis=1, keepdims=True))
    out_ref[...] = h / jnp.maximum(nrm, 1e-12) + b_ref[...]


_tc_finish = pl.pallas_call(
    _tc_body,
    grid=(N_NODES // _R,),
    in_specs=[
        pl.BlockSpec((_R, D), lambda i: (i, 0)),
        pl.BlockSpec((NC, _R, D), lambda i: (0, i, 0)),
        pl.BlockSpec((NC, _R, DW), lambda i: (0, i, 0)),
        pl.BlockSpec((2 * D, D), lambda i: (0, 0)),
        pl.BlockSpec((1, D), lambda i: (0, 0)),
    ],
    out_specs=pl.BlockSpec((_R, D), lambda i: (i, 0)),
    out_shape=jax.ShapeDtypeStruct((N_NODES, D), jnp.float32),
)


def kernel(x, edge_index, weight, bias):
    src = edge_index[0].reshape(NW, CH, K)
    dst = edge_index[1].reshape(NW, CH, K)
    psum, pdeg = _sc_aggregate(x, src, dst)
    return _tc_finish(x, psum, pdeg, weight, bias.reshape(1, D))


# speedup vs baseline: 13.3731x; 1.6087x over previous
import functools

import jax
import jax.numpy as jnp
from jax import lax
from jax.experimental import pallas as pl
from jax.experimental.pallas import tpu as pltpu
from jax.experimental.pallas import tpu_sc as plsc

N_NODES = 10000
N_EDGES = 320000
D = 128

NC = 2
NS = 16
NW = NC * NS
EPT = N_EDGES // NW
K = 80
CH = EPT // K
NPAD = 10240
RPT = NPAD // NS
DW = 8
NPAIR = CH // 2

_mesh = plsc.VectorSubcoreMesh(core_axis_name="c", subcore_axis_name="s")


@functools.partial(
    pl.kernel,
    out_type=[
        jax.ShapeDtypeStruct((NC, NPAD, D), jnp.float32),
        jax.ShapeDtypeStruct((NC, NPAD, DW), jnp.float32),
    ],
    mesh=_mesh,
    scratch_types=[
        pltpu.VMEM((CH, K), jnp.int32),
        pltpu.VMEM((CH, K), jnp.int32),
        pltpu.VMEM((2, K, D), jnp.float32),
        pltpu.VMEM((K, DW), jnp.float32),
        pltpu.VMEM((K, DW), jnp.float32),
        pltpu.VMEM_SHARED((NPAD, D), jnp.float32),
        pltpu.VMEM_SHARED((NPAD, DW), jnp.float32),
        pltpu.SemaphoreType.DMA,
        pltpu.SemaphoreType.DMA,
        pltpu.SemaphoreType.DMA,
        pltpu.SemaphoreType.DMA,
        pltpu.SemaphoreType.DMA,
        pltpu.SemaphoreType.DMA,
    ],
    compiler_params=pltpu.CompilerParams(use_tc_tiling_on_sc=False),
)
def _sc_aggregate(x_hbm, src_hbm, dst_hbm, out_sum, out_deg,
                  idx_s, idx_d, rows, ones_v, zdeg,
                  acc_sh, deg_sh, gsem0, gsem1, ssem0, ssem1, dsem0, dsem1):
    cid = lax.axis_index("c")
    sid = lax.axis_index("s")
    wid = cid * NS + sid

    z16 = jnp.zeros((16,), jnp.float32)
    o8 = jnp.ones((DW,), jnp.float32)
    z8 = jnp.zeros((DW,), jnp.float32)

    def _zrow(r, _):
        for c in range(D // 16):
            rows[0, r, pl.ds(c * 16, 16)] = z16
        ones_v[r, :] = o8
        zdeg[r, :] = z8
        return 0

    lax.fori_loop(0, K, _zrow, 0)

    for b in range(RPT // K):
        pltpu.sync_copy(rows.at[0], acc_sh.at[pl.ds(sid * RPT + b * K, K)])
        pltpu.sync_copy(zdeg, deg_sh.at[pl.ds(sid * RPT + b * K, K)])
    plsc.subcore_barrier()

    pltpu.sync_copy(src_hbm.at[wid], idx_s)
    pltpu.sync_copy(dst_hbm.at[wid], idx_d)

    gsems = (gsem0, gsem1)
    ssems = (ssem0, ssem1)
    dsems = (dsem0, dsem1)

    def _start_gather(j, b):
        pltpu.async_copy(x_hbm.at[idx_s.at[j]], rows.at[b], gsems[b])

    def _wait_gather(j, b):
        pltpu.make_async_copy(x_hbm.at[idx_s.at[j]], rows.at[b],
                              gsems[b]).wait()

    def _start_scatter(j, b):
        pltpu.async_copy(rows.at[b], acc_sh.at[idx_d.at[j]], ssems[b],
                         add=True)
        pltpu.async_copy(ones_v, deg_sh.at[idx_d.at[j]], dsems[b], add=True)

    def _wait_scatter(j, b):
        pltpu.make_async_copy(rows.at[b], acc_sh.at[idx_d.at[j]],
                              ssems[b]).wait()
        pltpu.make_async_copy(ones_v, deg_sh.at[idx_d.at[j]],
                              dsems[b]).wait()

    def _pair(t, peeled):
        j0 = 2 * t
        if not peeled:
            _wait_scatter(j0 - 1, 1)
        _start_gather(j0 + 1, 1)
        _wait_gather(j0, 0)
        _start_scatter(j0, 0)
        _wait_scatter(j0, 0)
        _start_gather(j0 + 2, 0)
        _wait_gather(j0 + 1, 1)
        _start_scatter(j0 + 1, 1)
        return 0

    _start_gather(0, 0)
    _pair(0, True)
    lax.fori_loop(1, NPAIR, lambda t, c: _pair(t, False), 0)
    _wait_scatter(CH - 2, 1)
    _wait_gather(CH - 1, 0)
    _start_scatter(CH - 1, 0)
    _wait_scatter(CH - 1, 0)
    plsc.subcore_barrier()

    pltpu.sync_copy(acc_sh.at[pl.ds(sid * RPT, RPT)],
                    out_sum.at[cid, pl.ds(sid * RPT, RPT)])
    pltpu.sync_copy(deg_sh.at[pl.ds(sid * RPT, RPT)],
                    out_deg.at[cid, pl.ds(sid * RPT, RPT)])


_R = 1000


def _tc_body(x_ref, ps_ref, pd_ref, w_ref, b_ref, out_ref):
    s = ps_ref[0] + ps_ref[1]
    d = pd_ref[0, :, 0:1] + pd_ref[1, :, 0:1]
    agg = s / jnp.maximum(d, 1.0)
    h = jnp.dot(x_ref[...], w_ref[0:D, :], preferred_element_type=jnp.float32)
    h += jnp.dot(agg, w_ref[D:2 * D, :], preferred_element_type=jnp.float32)
    nrm = jnp.sqrt(jnp.sum(h * h, axis=1, keepdims=True))
    out_ref[...] = h / jnp.maximum(nrm, 1e-12) + b_ref[...]


_tc_finish = pl.pallas_call(
    _tc_body,
    grid=(N_NODES // _R,),
    in_specs=[
        pl.BlockSpec((_R, D), lambda i: (i, 0)),
        pl.BlockSpec((NC, _R, D), lambda i: (0, i, 0)),
        pl.BlockSpec((NC, _R, DW), lambda i: (0, i, 0)),
        pl.BlockSpec((2 * D, D), lambda i: (0, 0)),
        pl.BlockSpec((1, D), lambda i: (0, 0)),
    ],
    out_specs=pl.BlockSpec((_R, D), lambda i: (i, 0)),
    out_shape=jax.ShapeDtypeStruct((N_NODES, D), jnp.float32),
)


def kernel(x, edge_index, weight, bias):
    src = edge_index[0].reshape(NW, CH, K)
    dst = edge_index[1].reshape(NW, CH, K)
    psum, pdeg = _sc_aggregate(x, src, dst)
    return _tc_finish(x, psum, pdeg, weight, bias.reshape(1, D))


# speedup vs baseline: 14.3456x; 1.0727x over previous
import functools

import jax
import jax.numpy as jnp
from jax import lax
from jax.experimental import pallas as pl
from jax.experimental.pallas import tpu as pltpu
from jax.experimental.pallas import tpu_sc as plsc

N_NODES = 10000
N_EDGES = 320000
D = 128

NC = 2
NS = 16
NW = NC * NS
EPT = N_EDGES // NW
K = 80
CH = EPT // K
NPAD = 10240
RPT = NPAD // NS
DW = 8
NPAIR = CH // 2

_mesh = plsc.VectorSubcoreMesh(core_axis_name="c", subcore_axis_name="s")


@functools.partial(
    pl.kernel,
    out_type=[
        jax.ShapeDtypeStruct((NC, NPAD, D), jnp.float32),
        jax.ShapeDtypeStruct((NC, NPAD, DW), jnp.float32),
    ],
    mesh=_mesh,
    scratch_types=[
        pltpu.VMEM((EPT,), jnp.int32),
        pltpu.VMEM((EPT,), jnp.int32),
        pltpu.VMEM((2, K, D), jnp.float32),
        pltpu.VMEM((K, DW), jnp.float32),
        pltpu.VMEM((K, DW), jnp.float32),
        pltpu.VMEM_SHARED((NPAD, D), jnp.float32),
        pltpu.VMEM_SHARED((NPAD, DW), jnp.float32),
        pltpu.SemaphoreType.DMA,
        pltpu.SemaphoreType.DMA,
        pltpu.SemaphoreType.DMA,
        pltpu.SemaphoreType.DMA,
        pltpu.SemaphoreType.DMA,
        pltpu.SemaphoreType.DMA,
    ],
    compiler_params=pltpu.CompilerParams(use_tc_tiling_on_sc=False),
)
def _sc_aggregate(x_hbm, ei_hbm, out_sum, out_deg,
                  idx_s, idx_d, rows, ones_v, zdeg,
                  acc_sh, deg_sh, gsem0, gsem1, ssem0, ssem1, dsem0, dsem1):
    cid = lax.axis_index("c")
    sid = lax.axis_index("s")
    wid = cid * NS + sid

    z16 = jnp.zeros((16,), jnp.float32)
    o8 = jnp.ones((DW,), jnp.float32)
    z8 = jnp.zeros((DW,), jnp.float32)

    def _zrow(r, _):
        for c in range(D // 16):
            rows[0, r, pl.ds(c * 16, 16)] = z16
        ones_v[r, :] = o8
        zdeg[r, :] = z8
        return 0

    lax.fori_loop(0, K, _zrow, 0)

    for b in range(RPT // K):
        pltpu.sync_copy(rows.at[0], acc_sh.at[pl.ds(sid * RPT + b * K, K)])
        pltpu.sync_copy(zdeg, deg_sh.at[pl.ds(sid * RPT + b * K, K)])
    plsc.subcore_barrier()

    pltpu.sync_copy(ei_hbm.at[0, pl.ds(wid * EPT, EPT)], idx_s)
    pltpu.sync_copy(ei_hbm.at[1, pl.ds(wid * EPT, EPT)], idx_d)

    gsems = (gsem0, gsem1)
    ssems = (ssem0, ssem1)
    dsems = (dsem0, dsem1)

    def _start_gather(j, b):
        pltpu.async_copy(x_hbm.at[idx_s.at[pl.ds(j * K, K)]], rows.at[b],
                         gsems[b])

    def _wait_gather(j, b):
        pltpu.make_async_copy(x_hbm.at[idx_s.at[pl.ds(j * K, K)]],
                              rows.at[b], gsems[b]).wait()

    def _start_scatter(j, b):
        pltpu.async_copy(rows.at[b], acc_sh.at[idx_d.at[pl.ds(j * K, K)]],
                         ssems[b], add=True)
        pltpu.async_copy(ones_v, deg_sh.at[idx_d.at[pl.ds(j * K, K)]],
                         dsems[b], add=True)

    def _wait_scatter(j, b):
        pltpu.make_async_copy(rows.at[b], acc_sh.at[idx_d.at[pl.ds(j * K, K)]],
                              ssems[b]).wait()
        pltpu.make_async_copy(ones_v, deg_sh.at[idx_d.at[pl.ds(j * K, K)]],
                              dsems[b]).wait()

    def _pair(t, peeled):
        j0 = 2 * t
        if not peeled:
            _wait_scatter(j0 - 1, 1)
        _start_gather(j0 + 1, 1)
        _wait_gather(j0, 0)
        _start_scatter(j0, 0)
        _wait_scatter(j0, 0)
        _start_gather(j0 + 2, 0)
        _wait_gather(j0 + 1, 1)
        _start_scatter(j0 + 1, 1)
        return 0

    _start_gather(0, 0)
    _pair(0, True)
    lax.fori_loop(1, NPAIR, lambda t, c: _pair(t, False), 0)
    _wait_scatter(CH - 2, 1)
    _wait_gather(CH - 1, 0)
    _start_scatter(CH - 1, 0)
    _wait_scatter(CH - 1, 0)
    plsc.subcore_barrier()

    pltpu.sync_copy(acc_sh.at[pl.ds(sid * RPT, RPT)],
                    out_sum.at[cid, pl.ds(sid * RPT, RPT)])
    pltpu.sync_copy(deg_sh.at[pl.ds(sid * RPT, RPT)],
                    out_deg.at[cid, pl.ds(sid * RPT, RPT)])


_R = 2000


def _tc_body(x_ref, ps_ref, pd_ref, w_ref, b_ref, out_ref):
    s = ps_ref[0] + ps_ref[1]
    d = pd_ref[0, :, 0:1] + pd_ref[1, :, 0:1]
    agg = s / jnp.maximum(d, 1.0)
    h = jnp.dot(x_ref[...], w_ref[0:D, :], preferred_element_type=jnp.float32)
    h += jnp.dot(agg, w_ref[D:2 * D, :], preferred_element_type=jnp.float32)
    nrm = jnp.sqrt(jnp.sum(h * h, axis=1, keepdims=True))
    out_ref[...] = h / jnp.maximum(nrm, 1e-12) + b_ref[...]


_tc_finish = pl.pallas_call(
    _tc_body,
    grid=(N_NODES // _R,),
    in_specs=[
        pl.BlockSpec((_R, D), lambda i: (i, 0)),
        pl.BlockSpec((NC, _R, D), lambda i: (0, i, 0)),
        pl.BlockSpec((NC, _R, DW), lambda i: (0, i, 0)),
        pl.BlockSpec((2 * D, D), lambda i: (0, 0)),
        pl.BlockSpec((1, D), lambda i: (0, 0)),
    ],
    out_specs=pl.BlockSpec((_R, D), lambda i: (i, 0)),
    out_shape=jax.ShapeDtypeStruct((N_NODES, D), jnp.float32),
)


def kernel(x, edge_index, weight, bias):
    psum, pdeg = _sc_aggregate(x, edge_index)
    return _tc_finish(x, psum, pdeg, weight, bias.reshape(1, D))


# speedup vs baseline: 15.8396x; 1.1041x over previous
import functools

import jax
import jax.numpy as jnp
from jax import lax
from jax.experimental import pallas as pl
from jax.experimental.pallas import tpu as pltpu
from jax.experimental.pallas import tpu_sc as plsc

N_NODES = 10000
N_EDGES = 320000
D = 128

NC = 2
NS = 16
NW = NC * NS
EPT = N_EDGES // NW
K = 40
CH = EPT // K
NB = 4
NGRP = CH // NB
NPAD = 10240
RPT = NPAD // NS
DW = 8

_mesh = plsc.VectorSubcoreMesh(core_axis_name="c", subcore_axis_name="s")


@functools.partial(
    pl.kernel,
    out_type=[
        jax.ShapeDtypeStruct((NC, NPAD, D), jnp.float32),
        jax.ShapeDtypeStruct((NC, NPAD, DW), jnp.float32),
    ],
    mesh=_mesh,
    scratch_types=[
        pltpu.VMEM((EPT,), jnp.int32),
        pltpu.VMEM((EPT,), jnp.int32),
        pltpu.VMEM((NB, K, D), jnp.float32),
        pltpu.VMEM((K, DW), jnp.float32),
        pltpu.VMEM((K, DW), jnp.float32),
        pltpu.VMEM_SHARED((NPAD, D), jnp.float32),
        pltpu.VMEM_SHARED((NPAD, DW), jnp.float32),
    ] + [pltpu.SemaphoreType.DMA] * (3 * NB),
    compiler_params=pltpu.CompilerParams(use_tc_tiling_on_sc=False),
)
def _sc_aggregate(x_hbm, ei_hbm, out_sum, out_deg,
                  idx_s, idx_d, rows, ones_v, zdeg,
                  acc_sh, deg_sh, *sems):
    cid = lax.axis_index("c")
    sid = lax.axis_index("s")
    wid = cid * NS + sid

    z16 = jnp.zeros((16,), jnp.float32)
    o8 = jnp.ones((DW,), jnp.float32)
    z8 = jnp.zeros((DW,), jnp.float32)

    def _zrow(r, _):
        for c in range(D // 16):
            rows[0, r, pl.ds(c * 16, 16)] = z16
        ones_v[r, :] = o8
        zdeg[r, :] = z8
        return 0

    lax.fori_loop(0, K, _zrow, 0)

    for b in range(RPT // K):
        pltpu.sync_copy(rows.at[0], acc_sh.at[pl.ds(sid * RPT + b * K, K)])
        pltpu.sync_copy(zdeg, deg_sh.at[pl.ds(sid * RPT + b * K, K)])
    plsc.subcore_barrier()

    pltpu.sync_copy(ei_hbm.at[0, pl.ds(wid * EPT, EPT)], idx_s)
    pltpu.sync_copy(ei_hbm.at[1, pl.ds(wid * EPT, EPT)], idx_d)

    gsems = sems[0:NB]
    ssems = sems[NB:2 * NB]
    dsems = sems[2 * NB:3 * NB]

    def _start_gather(j, b):
        pltpu.async_copy(x_hbm.at[idx_s.at[pl.ds(j * K, K)]], rows.at[b],
                         gsems[b])

    def _wait_gather(j, b):
        pltpu.make_async_copy(x_hbm.at[idx_s.at[pl.ds(j * K, K)]],
                              rows.at[b], gsems[b]).wait()

    def _start_scatter(j, b):
        pltpu.async_copy(rows.at[b], acc_sh.at[idx_d.at[pl.ds(j * K, K)]],
                         ssems[b], add=True)
        pltpu.async_copy(ones_v, deg_sh.at[idx_d.at[pl.ds(j * K, K)]],
                         dsems[b], add=True)

    def _wait_scatter(j, b):
        pltpu.make_async_copy(rows.at[b], acc_sh.at[idx_d.at[pl.ds(j * K, K)]],
                              ssems[b]).wait()
        pltpu.make_async_copy(ones_v, deg_sh.at[idx_d.at[pl.ds(j * K, K)]],
                              dsems[b]).wait()

    def _grp(t, peeled):
        c0 = NB * t
        for i in range(NB):
            j = c0 + i
            jn = j + 2
            bn = (i + 2) % NB
            if not peeled or i >= 2:
                _wait_scatter(jn - NB, bn)
            _start_gather(jn, bn)
            _wait_gather(j, i)
            _start_scatter(j, i)
        return 0

    _start_gather(0, 0)
    _start_gather(1, 1)
    _grp(0, True)
    lax.fori_loop(1, NGRP, lambda t, c: _grp(t, False), 0)
    _wait_scatter(CH - 4, (CH - 4) % NB)
    _wait_scatter(CH - 3, (CH - 3) % NB)
    for j in (CH - 2, CH - 1):
        _wait_gather(j, j % NB)
        _start_scatter(j, j % NB)
    for j in (CH - 2, CH - 1):
        _wait_scatter(j, j % NB)
    plsc.subcore_barrier()

    pltpu.sync_copy(acc_sh.at[pl.ds(sid * RPT, RPT)],
                    out_sum.at[cid, pl.ds(sid * RPT, RPT)])
    pltpu.sync_copy(deg_sh.at[pl.ds(sid * RPT, RPT)],
                    out_deg.at[cid, pl.ds(sid * RPT, RPT)])


_R = 2000


def _tc_body(x_ref, ps_ref, pd_ref, w_ref, b_ref, out_ref):
    s = ps_ref[0] + ps_ref[1]
    d = pd_ref[0, :, 0:1] + pd_ref[1, :, 0:1]
    agg = s / jnp.maximum(d, 1.0)
    h = jnp.dot(x_ref[...], w_ref[0:D, :], preferred_element_type=jnp.float32)
    h += jnp.dot(agg, w_ref[D:2 * D, :], preferred_element_type=jnp.float32)
    nrm = jnp.sqrt(jnp.sum(h * h, axis=1, keepdims=True))
    out_ref[...] = h / jnp.maximum(nrm, 1e-12) + b_ref[...]


_tc_finish = pl.pallas_call(
    _tc_body,
    grid=(N_NODES // _R,),
    in_specs=[
        pl.BlockSpec((_R, D), lambda i: (i, 0)),
        pl.BlockSpec((NC, _R, D), lambda i: (0, i, 0)),
        pl.BlockSpec((NC, _R, DW), lambda i: (0, i, 0)),
        pl.BlockSpec((2 * D, D), lambda i: (0, 0)),
        pl.BlockSpec((1, D), lambda i: (0, 0)),
    ],
    out_specs=pl.BlockSpec((_R, D), lambda i: (i, 0)),
    out_shape=jax.ShapeDtypeStruct((N_NODES, D), jnp.float32),
)


def kernel(x, edge_index, weight, bias):
    psum, pdeg = _sc_aggregate(x, edge_index)
    return _tc_finish(x, psum, pdeg, weight, bias.reshape(1, D))


# speedup vs baseline: 17.1155x; 1.0806x over previous
import functools

import jax
import jax.numpy as jnp
from jax import lax
from jax.experimental import pallas as pl
from jax.experimental.pallas import tpu as pltpu
from jax.experimental.pallas import tpu_sc as plsc

N_NODES = 10000
N_EDGES = 320000
D = 128

NC = 2
NS = 16
NW = NC * NS
EPT = N_EDGES // NW
K = 40
CH = EPT // K
NB = 4
NGRP = CH // NB
NPAD = 10240
RPT = NPAD // NS
DW = 8

_mesh = plsc.VectorSubcoreMesh(core_axis_name="c", subcore_axis_name="s")


@functools.partial(
    pl.kernel,
    out_type=[
        jax.ShapeDtypeStruct((NC, NPAD, D), jnp.float32),
        jax.ShapeDtypeStruct((NC, NPAD, D), jnp.float32),
    ],
    mesh=_mesh,
    scratch_types=[
        pltpu.VMEM((EPT,), jnp.int32),
        pltpu.VMEM((EPT,), jnp.int32),
        pltpu.VMEM((NB, K, D), jnp.float32),
        pltpu.VMEM((K, DW), jnp.float32),
        pltpu.VMEM((K, DW), jnp.float32),
        pltpu.VMEM_SHARED((NPAD, D), jnp.float32),
        pltpu.VMEM_SHARED((NPAD, DW), jnp.float32),
    ] + [pltpu.SemaphoreType.DMA] * (3 * NB),
    compiler_params=pltpu.CompilerParams(use_tc_tiling_on_sc=False),
)
def _sc_aggregate(x_hbm, ei_hbm, out_sum, out_deg,
                  idx_s, idx_d, rows, ones_v, zdeg,
                  acc_sh, deg_sh, *sems):
    cid = lax.axis_index("c")
    sid = lax.axis_index("s")
    wid = cid * NS + sid

    z16 = jnp.zeros((16,), jnp.float32)
    o8 = jnp.ones((DW,), jnp.float32)
    z8 = jnp.zeros((DW,), jnp.float32)

    def _zrow(r, _):
        for c in range(D // 16):
            rows[0, r, pl.ds(c * 16, 16)] = z16
        ones_v[r, :] = o8
        zdeg[r, :] = z8
        return 0

    lax.fori_loop(0, K, _zrow, 0)

    gsems = sems[0:NB]
    ssems = sems[NB:2 * NB]
    dsems = sems[2 * NB:3 * NB]

    zw = []
    for b in range(RPT // K):
        zw.append(pltpu.async_copy(
            rows.at[0], acc_sh.at[pl.ds(sid * RPT + b * K, K)],
            gsems[b % 2]))
        zw.append(pltpu.async_copy(
            zdeg, deg_sh.at[pl.ds(sid * RPT + b * K, K)],
            dsems[b % 2]))
    zw.append(pltpu.async_copy(ei_hbm.at[0, pl.ds(wid * EPT, EPT)], idx_s,
                               gsems[2]))
    zw.append(pltpu.async_copy(ei_hbm.at[1, pl.ds(wid * EPT, EPT)], idx_d,
                               gsems[3]))
    for h in zw:
        h.wait()
    plsc.subcore_barrier()

    def _start_gather(j, b):
        pltpu.async_copy(x_hbm.at[idx_s.at[pl.ds(j * K, K)]], rows.at[b],
                         gsems[b])

    def _wait_gather(j, b):
        pltpu.make_async_copy(x_hbm.at[idx_s.at[pl.ds(j * K, K)]],
                              rows.at[b], gsems[b]).wait()

    def _start_scatter(j, b):
        pltpu.async_copy(rows.at[b], acc_sh.at[idx_d.at[pl.ds(j * K, K)]],
                         ssems[b], add=True)
        pltpu.async_copy(ones_v, deg_sh.at[idx_d.at[pl.ds(j * K, K)]],
                         dsems[b], add=True)

    def _wait_scatter(j, b):
        pltpu.make_async_copy(rows.at[b], acc_sh.at[idx_d.at[pl.ds(j * K, K)]],
                              ssems[b]).wait()
        pltpu.make_async_copy(ones_v, deg_sh.at[idx_d.at[pl.ds(j * K, K)]],
                              dsems[b]).wait()

    def _grp(t, peeled):
        c0 = NB * t
        for i in range(NB):
            j = c0 + i
            jn = j + 2
            bn = (i + 2) % NB
            if not peeled or i >= 2:
                _wait_scatter(jn - NB, bn)
            _start_gather(jn, bn)
            _wait_gather(j, i)
            _start_scatter(j, i)
        return 0

    _start_gather(0, 0)
    _start_gather(1, 1)
    _grp(0, True)
    lax.fori_loop(1, NGRP, lambda t, c: _grp(t, False), 0)
    _wait_scatter(CH - 4, (CH - 4) % NB)
    _wait_scatter(CH - 3, (CH - 3) % NB)
    for j in (CH - 2, CH - 1):
        _wait_gather(j, j % NB)
        _start_scatter(j, j % NB)
    for j in (CH - 2, CH - 1):
        _wait_scatter(j, j % NB)
    plsc.subcore_barrier()

    h1 = pltpu.async_copy(acc_sh.at[pl.ds(sid * RPT, RPT)],
                          out_sum.at[cid, pl.ds(sid * RPT, RPT)], gsems[0])
    h2 = pltpu.async_copy(deg_sh.at[pl.ds(sid * RPT, RPT)],
                          out_deg.at[cid, pl.ds(sid * RPT, RPT), pl.ds(0, DW)],
                          gsems[1])
    h1.wait()
    h2.wait()


_R = 2000


def _tc_body(x_ref, ps_ref, pd_ref, w_ref, b_ref, out_ref):
    s = ps_ref[0] + ps_ref[1]
    d = pd_ref[0, :, 0:1] + pd_ref[1, :, 0:1]
    agg = s / jnp.maximum(d, 1.0)
    h = jnp.dot(x_ref[...], w_ref[0:D, :], preferred_element_type=jnp.float32)
    h += jnp.dot(agg, w_ref[D:2 * D, :], preferred_element_type=jnp.float32)
    nrm = jnp.sqrt(jnp.sum(h * h, axis=1, keepdims=True))
    out_ref[...] = h / jnp.maximum(nrm, 1e-12) + b_ref[...]


_tc_finish = pl.pallas_call(
    _tc_body,
    grid=(N_NODES // _R,),
    in_specs=[
        pl.BlockSpec((_R, D), lambda i: (i, 0)),
        pl.BlockSpec((NC, _R, D), lambda i: (0, i, 0)),
        pl.BlockSpec((NC, _R, D), lambda i: (0, i, 0)),
        pl.BlockSpec((2 * D, D), lambda i: (0, 0)),
        pl.BlockSpec((1, D), lambda i: (0, 0)),
    ],
    out_specs=pl.BlockSpec((_R, D), lambda i: (i, 0)),
    out_shape=jax.ShapeDtypeStruct((N_NODES, D), jnp.float32),
)


def kernel(x, edge_index, weight, bias):
    psum, pdeg = _sc_aggregate(x, edge_index)
    return _tc_finish(x, psum, pdeg, weight, bias.reshape(1, D))
